# baseline (speedup 1.0000x reference)
import functools
import jax
import jax.numpy as jnp
from jax import lax
from jax.experimental import pallas as pl
from jax.experimental.pallas import tpu as pltpu
from jax.experimental.pallas import tpu_sc as plsc

M = 65
N = 100000
F = 128
NUM_HEAD = 8
SLOPE = 0.2
INT_MAX = jnp.iinfo(jnp.int32).max

NS = 16
L = 16
CH = 6256
CSTEPS = CH // L
UNROLL = 17
OUTER = CSTEPS // UNROLL
LAST_OFF = N - CH
ROWS0 = 33


def _sc_body(adj_hbm, x_hbm, sel_hbm, buf, pm_v, shared_pm, allpm, rebuf,
             idx_v, rows_v, sem0, sem1, gsem):
    c = lax.axis_index("c")
    s = lax.axis_index("s")
    rbase = c * ROWS0
    nrows = ROWS0 - c
    offs = jnp.minimum(s * CH, LAST_OFF)
    lanes = lax.broadcasted_iota(jnp.int32, (L,), 0)
    sems = (sem0, sem1)

    def copy_row(r):
        return pltpu.make_async_copy(
            adj_hbm.at[rbase + r, pl.ds(offs, CH)],
            buf.at[r % 2], sems[r % 2])

    copy_row(0).start()
    copy_row(1).start()

    pms = [jnp.full((L,), -1.0, jnp.float32) for _ in range(3)]
    for r in range(ROWS0):
        @pl.when(r < nrows)
        def _wait():
            copy_row(r).wait()
        if r + 2 < ROWS0:
            @pl.when(r + 2 < nrows)
            def _next():
                copy_row(r + 2).start()

        bref = buf.at[r % 2]

        def step(o, vm):
            for u in range(UNROLL):
                vm = jnp.maximum(vm, bref[pl.ds((o * UNROLL + u) * L, L)])
            return vm
        vm = lax.fori_loop(0, OUTER, step,
                           jnp.full((L,), -1.0, jnp.float32))
        m_r = jnp.max(vm)
        g, l = r // L, r % L
        pms[g] = jnp.where(lanes == l, m_r, pms[g])

    for g in range(3):
        pm_v[pl.ds(g * L, L)] = pms[g]

    pltpu.sync_copy(pm_v, shared_pm.at[s])
    plsc.subcore_barrier()
    pltpu.sync_copy(shared_pm, allpm)

    idxvec = jnp.zeros((L,), jnp.int32)
    for jj in range(3):
        q = 2 * s + jj
        valid = (q < nrows) & ((jj < 2) | (s == NS - 1))
        qc = jnp.minimum(q, nrows - 1)
        mvs = [jnp.full((L,), -1.0, jnp.float32) for _ in range(3)]
        wvs = [jnp.zeros((L,), jnp.int32) for _ in range(3)]
        for t in range(NS):
            for g in range(3):
                v = allpm[t, pl.ds(g * L, L)]
                better = v > mvs[g]
                mvs[g] = jnp.where(better, v, mvs[g])
                wvs[g] = jnp.where(better, jnp.int32(t), wvs[g])
        qg = qc // L
        qlane = qc % L
        mv = jnp.where(qg == 0, mvs[0], jnp.where(qg == 1, mvs[1], mvs[2]))
        wv = jnp.where(qg == 0, wvs[0], jnp.where(qg == 1, wvs[1], wvs[2]))
        sel_lane = lanes == qlane
        m = jnp.sum(jnp.where(sel_lane, mv, 0.0))
        win = jnp.sum(jnp.where(sel_lane, wv, 0))
        woffs = jnp.minimum(win * CH, LAST_OFF)
        pltpu.sync_copy(adj_hbm.at[rbase + qc, pl.ds(woffs, CH)], rebuf)
        mvec = jnp.full((L,), m)

        def fstep(o, rm):
            for u in range(UNROLL):
                k = (o * UNROLL + u) * L
                v = rebuf[pl.ds(k, L)]
                rm = jnp.minimum(
                    rm, jnp.where(v == mvec, k + lanes, INT_MAX))
            return rm
        rmin = lax.fori_loop(0, OUTER, fstep,
                             jnp.full((L,), INT_MAX, jnp.int32))
        gidx = woffs + jnp.min(rmin)
        idxvec = jnp.where(valid & (lanes == jj), gidx, idxvec)

    idx_v[...] = idxvec
    pltpu.async_copy(x_hbm.at[idx_v], rows_v, gsem).wait()
    for jj in range(3):
        q = 2 * s + jj
        valid = (q < nrows) & ((jj < 2) | (s == NS - 1))

        @pl.when(valid)
        def _emit():
            pltpu.sync_copy(rows_v.at[pl.ds(jj, 1), :],
                            sel_hbm.at[pl.ds(rbase + q, 1), :])


_sc_gather = functools.partial(
    pl.kernel,
    mesh=plsc.VectorSubcoreMesh(core_axis_name="c", subcore_axis_name="s"),
    compiler_params=pltpu.CompilerParams(use_tc_tiling_on_sc=False, needs_layout_passes=False),
    out_type=jax.ShapeDtypeStruct((M, F), jnp.float32),
    scratch_types=[
        pltpu.VMEM((2, CH), jnp.float32),
        pltpu.VMEM((3 * L,), jnp.float32),
        pltpu.VMEM_SHARED((NS, 3 * L), jnp.float32),
        pltpu.VMEM((NS, 3 * L), jnp.float32),
        pltpu.VMEM((CH,), jnp.float32),
        pltpu.VMEM((L,), jnp.int32),
        pltpu.VMEM((L, F), jnp.float32),
        pltpu.SemaphoreType.DMA,
        pltpu.SemaphoreType.DMA,
        pltpu.SemaphoreType.DMA,
    ],
)(_sc_body)


def _gat_body(sel_ref, w_ref, a_ref, b_ref, out_ref):
    sel = sel_ref[...]
    h = jnp.dot(sel, w_ref[...], preferred_element_type=jnp.float32)
    a0 = a_ref[0:1, :]
    a1 = a_ref[1:2, :]
    c = jnp.sum(h[0:1, :] * a0)
    d = jnp.sum(h * a1, axis=1, keepdims=True)
    lg = c + d
    lg = jnp.where(lg >= 0, lg, SLOPE * lg)
    ridx = jax.lax.broadcasted_iota(jnp.int32, (M, 1), 0)
    e = jnp.where(ridx >= 1, jnp.exp(lg), 0.0)
    alpha = e / jnp.sum(e)
    hp = jnp.sum(alpha * h, axis=0, keepdims=True) + b_ref[...]
    out_ref[...] = jnp.broadcast_to(hp, (NUM_HEAD, F))


def kernel(x, adj, weight, a, bias):
    sel = _sc_gather(adj, x)
    out = pl.pallas_call(
        _gat_body,
        out_shape=jax.ShapeDtypeStruct((NUM_HEAD, F), jnp.float32),
    )(sel, weight, a.reshape(2, F), bias.reshape(1, F))
    return out.reshape(NUM_HEAD * F)


# speedup vs baseline: 1.0034x; 1.0034x over previous
import functools
import jax
import jax.numpy as jnp
from jax import lax
from jax.experimental import pallas as pl
from jax.experimental.pallas import tpu as pltpu
from jax.experimental.pallas import tpu_sc as plsc

M = 65
N = 100000
F = 128
NUM_HEAD = 8
SLOPE = 0.2
INT_MAX = jnp.iinfo(jnp.int32).max

NS = 16
L = 16
CH = 6256
CSTEPS = CH // L
UNROLL = 17
OUTER = CSTEPS // UNROLL
LAST_OFF = N - CH
ROWS0 = 33


def _sc_body(adj_hbm, x_hbm, sel_hbm, buf, pm_v, shared_pm, allpm, rebuf,
             idx_v, rows_v, sem0, sem1, gsem):
    c = lax.axis_index("c")
    s = lax.axis_index("s")
    rbase = c * ROWS0
    nrows = ROWS0 - c
    offs = jnp.minimum(s * CH, LAST_OFF)
    lanes = lax.broadcasted_iota(jnp.int32, (L,), 0)
    sems = (sem0, sem1)

    def copy_row(r):
        return pltpu.make_async_copy(
            adj_hbm.at[rbase + r, pl.ds(offs, CH)],
            buf.at[r % 2], sems[r % 2])

    copy_row(0).start()
    copy_row(1).start()

    pms = [jnp.full((L,), -1.0, jnp.float32) for _ in range(3)]
    for r in range(ROWS0):
        @pl.when(r < nrows)
        def _wait():
            copy_row(r).wait()
        if r + 2 < ROWS0:
            @pl.when(r + 2 < nrows)
            def _next():
                copy_row(r + 2).start()

        bref = buf.at[r % 2]

        def step(o, vms):
            vms = list(vms)
            for u in range(UNROLL):
                vms[u % 4] = jnp.maximum(
                    vms[u % 4], bref[pl.ds((o * UNROLL + u) * L, L)])
            return tuple(vms)
        init = tuple(jnp.full((L,), -1.0, jnp.float32) for _ in range(4))
        vms = lax.fori_loop(0, OUTER, step, init)
        vm = jnp.maximum(jnp.maximum(vms[0], vms[1]),
                         jnp.maximum(vms[2], vms[3]))
        m_r = jnp.max(vm)
        g, l = r // L, r % L
        pms[g] = jnp.where(lanes == l, m_r, pms[g])

    for g in range(3):
        pm_v[pl.ds(g * L, L)] = pms[g]

    pltpu.sync_copy(pm_v, shared_pm.at[s])
    plsc.subcore_barrier()
    pltpu.sync_copy(shared_pm, allpm)

    idxvec = jnp.zeros((L,), jnp.int32)
    for jj in range(3):
        q = 2 * s + jj
        valid = (q < nrows) & ((jj < 2) | (s == NS - 1))
        qc = jnp.minimum(q, nrows - 1)
        mvs = [jnp.full((L,), -1.0, jnp.float32) for _ in range(3)]
        wvs = [jnp.zeros((L,), jnp.int32) for _ in range(3)]
        for t in range(NS):
            for g in range(3):
                v = allpm[t, pl.ds(g * L, L)]
                better = v > mvs[g]
                mvs[g] = jnp.where(better, v, mvs[g])
                wvs[g] = jnp.where(better, jnp.int32(t), wvs[g])
        qg = qc // L
        qlane = qc % L
        mv = jnp.where(qg == 0, mvs[0], jnp.where(qg == 1, mvs[1], mvs[2]))
        wv = jnp.where(qg == 0, wvs[0], jnp.where(qg == 1, wvs[1], wvs[2]))
        sel_lane = lanes == qlane
        m = jnp.sum(jnp.where(sel_lane, mv, 0.0))
        win = jnp.sum(jnp.where(sel_lane, wv, 0))
        woffs = jnp.minimum(win * CH, LAST_OFF)
        pltpu.sync_copy(adj_hbm.at[rbase + qc, pl.ds(woffs, CH)], rebuf)
        mvec = jnp.full((L,), m)

        def fstep(o, rm):
            for u in range(UNROLL):
                k = (o * UNROLL + u) * L
                v = rebuf[pl.ds(k, L)]
                rm = jnp.minimum(
                    rm, jnp.where(v == mvec, k + lanes, INT_MAX))
            return rm
        rmin = lax.fori_loop(0, OUTER, fstep,
                             jnp.full((L,), INT_MAX, jnp.int32))
        gidx = woffs + jnp.min(rmin)
        idxvec = jnp.where(valid & (lanes == jj), gidx, idxvec)

    idx_v[...] = idxvec
    pltpu.async_copy(x_hbm.at[idx_v], rows_v, gsem).wait()
    for jj in range(3):
        q = 2 * s + jj
        valid = (q < nrows) & ((jj < 2) | (s == NS - 1))

        @pl.when(valid)
        def _emit():
            pltpu.sync_copy(rows_v.at[pl.ds(jj, 1), :],
                            sel_hbm.at[pl.ds(rbase + q, 1), :])


_sc_gather = functools.partial(
    pl.kernel,
    mesh=plsc.VectorSubcoreMesh(core_axis_name="c", subcore_axis_name="s"),
    compiler_params=pltpu.CompilerParams(use_tc_tiling_on_sc=False, needs_layout_passes=False),
    out_type=jax.ShapeDtypeStruct((M, F), jnp.float32),
    scratch_types=[
        pltpu.VMEM((2, CH), jnp.float32),
        pltpu.VMEM((3 * L,), jnp.float32),
        pltpu.VMEM_SHARED((NS, 3 * L), jnp.float32),
        pltpu.VMEM((NS, 3 * L), jnp.float32),
        pltpu.VMEM((CH,), jnp.float32),
        pltpu.VMEM((L,), jnp.int32),
        pltpu.VMEM((L, F), jnp.float32),
        pltpu.SemaphoreType.DMA,
        pltpu.SemaphoreType.DMA,
        pltpu.SemaphoreType.DMA,
    ],
)(_sc_body)


def _gat_body(sel_ref, w_ref, a_ref, b_ref, out_ref):
    sel = sel_ref[...]
    h = jnp.dot(sel, w_ref[...], preferred_element_type=jnp.float32)
    a0 = a_ref[0:1, :]
    a1 = a_ref[1:2, :]
    c = jnp.sum(h[0:1, :] * a0)
    d = jnp.sum(h * a1, axis=1, keepdims=True)
    lg = c + d
    lg = jnp.where(lg >= 0, lg, SLOPE * lg)
    ridx = jax.lax.broadcasted_iota(jnp.int32, (M, 1), 0)
    e = jnp.where(ridx >= 1, jnp.exp(lg), 0.0)
    alpha = e / jnp.sum(e)
    hp = jnp.sum(alpha * h, axis=0, keepdims=True) + b_ref[...]
    out_ref[...] = jnp.broadcast_to(hp, (NUM_HEAD, F))


def kernel(x, adj, weight, a, bias):
    sel = _sc_gather(adj, x)
    out = pl.pallas_call(
        _gat_body,
        out_shape=jax.ShapeDtypeStruct((NUM_HEAD, F), jnp.float32),
    )(sel, weight, a.reshape(2, F), bias.reshape(1, F))
    return out.reshape(NUM_HEAD * F)


# speedup vs baseline: 5.2658x; 5.2479x over previous
import jax
import jax.numpy as jnp
from jax.experimental import pallas as pl
from jax.experimental.pallas import tpu as pltpu

M = 65
N = 100000
F = 128
NUM_HEAD = 8
SLOPE = 0.2
BLK = 25088
NBLK = (N + BLK - 1) // BLK
INT_MAX = jnp.iinfo(jnp.int32).max


def _scan_body(adj_ref, idx_out_ref, max_sc, idx_sc):
    j = pl.program_id(0)

    @pl.when(j == 0)
    def _init():
        max_sc[...] = jnp.full((M, 1), -1.0, jnp.float32)
        idx_sc[...] = jnp.zeros((M, 1), jnp.int32)

    cols = j * BLK + jax.lax.broadcasted_iota(jnp.int32, (M, BLK), 1)
    vals = jnp.where(cols < N, adj_ref[...], -1.0)
    bmax = jnp.max(vals, axis=1, keepdims=True)
    bidx = jnp.min(jnp.where(vals == bmax, cols, INT_MAX),
                   axis=1, keepdims=True)
    better = bmax > max_sc[...]
    max_sc[...] = jnp.where(better, bmax, max_sc[...])
    idx_sc[...] = jnp.where(better, bidx, idx_sc[...])

    @pl.when(j == NBLK - 1)
    def _fin():
        idx_out_ref[...] = idx_sc[...]


def _gat_body(idx_ref, x_ref, w_ref, a_ref, b_ref, out_ref, rows_sc, sem):
    for i in range(M):
        pltpu.make_async_copy(
            x_ref.at[pl.ds(idx_ref[i], 1), :],
            rows_sc.at[pl.ds(i, 1), :], sem).start()
    for i in range(M):
        pltpu.make_async_copy(
            x_ref.at[pl.ds(0, 1), :],
            rows_sc.at[pl.ds(i, 1), :], sem).wait()

    sel = rows_sc[...]
    h = jnp.dot(sel, w_ref[...], preferred_element_type=jnp.float32)
    a0 = a_ref[0:1, :]
    a1 = a_ref[1:2, :]
    c = jnp.sum(h[0:1, :] * a0)
    d = jnp.sum(h * a1, axis=1, keepdims=True)
    lg = c + d
    lg = jnp.where(lg >= 0, lg, SLOPE * lg)
    ridx = jax.lax.broadcasted_iota(jnp.int32, (M, 1), 0)
    e = jnp.where(ridx >= 1, jnp.exp(lg), 0.0)
    alpha = e / jnp.sum(e)
    hp = jnp.sum(alpha * h, axis=0, keepdims=True) + b_ref[...]
    out_ref[...] = jnp.broadcast_to(hp, (NUM_HEAD, F))


def kernel(x, adj, weight, a, bias):
    idx2 = pl.pallas_call(
        _scan_body,
        grid=(NBLK,),
        in_specs=[pl.BlockSpec((M, BLK), lambda j: (0, j))],
        out_specs=pl.BlockSpec((M, 1), lambda j: (0, 0)),
        out_shape=jax.ShapeDtypeStruct((M, 1), jnp.int32),
        scratch_shapes=[pltpu.VMEM((M, 1), jnp.float32),
                        pltpu.VMEM((M, 1), jnp.int32)],
    )(adj)
    idx = idx2.reshape(M)

    out = pl.pallas_call(
        _gat_body,
        grid_spec=pltpu.PrefetchScalarGridSpec(
            num_scalar_prefetch=1,
            grid=(1,),
            in_specs=[
                pl.BlockSpec(memory_space=pl.ANY),
                pl.BlockSpec((F, F), lambda i, idx_ref: (0, 0)),
                pl.BlockSpec((2, F), lambda i, idx_ref: (0, 0)),
                pl.BlockSpec((1, F), lambda i, idx_ref: (0, 0)),
            ],
            out_specs=pl.BlockSpec((NUM_HEAD, F), lambda i, idx_ref: (0, 0)),
            scratch_shapes=[pltpu.VMEM((M, F), jnp.float32),
                            pltpu.SemaphoreType.DMA],
        ),
        out_shape=jax.ShapeDtypeStruct((NUM_HEAD, F), jnp.float32),
    )(idx, x, weight, a.reshape(2, F), bias.reshape(1, F))
    return out.reshape(NUM_HEAD * F)


# speedup vs baseline: 5.3584x; 1.0176x over previous
import jax
import jax.numpy as jnp
from jax.experimental import pallas as pl
from jax.experimental.pallas import tpu as pltpu

M = 65
N = 100000
F = 128
NUM_HEAD = 8
SLOPE = 0.2
BLK = 25088
NBLK = (N + BLK - 1) // BLK
INT_MAX = jnp.iinfo(jnp.int32).max


def _scan_body(adj_ref, idx_out_ref, max_sc, idx_sc):
    j = pl.program_id(0)

    @pl.when(j == 0)
    def _init():
        max_sc[...] = jnp.full((M, 1), -1.0, jnp.float32)
        idx_sc[...] = jnp.zeros((M, 1), jnp.int32)

    def _update(vals):
        cols = j * BLK + jax.lax.broadcasted_iota(jnp.int32, (M, BLK), 1)
        bmax = jnp.max(vals, axis=1, keepdims=True)
        bidx = jnp.min(jnp.where(vals == bmax, cols, INT_MAX),
                       axis=1, keepdims=True)
        better = bmax > max_sc[...]
        max_sc[...] = jnp.where(better, bmax, max_sc[...])
        idx_sc[...] = jnp.where(better, bidx, idx_sc[...])

    @pl.when(j < NBLK - 1)
    def _full():
        _update(adj_ref[...])

    @pl.when(j == NBLK - 1)
    def _tail():
        cols = j * BLK + jax.lax.broadcasted_iota(jnp.int32, (M, BLK), 1)
        _update(jnp.where(cols < N, adj_ref[...], -1.0))

    @pl.when(j == NBLK - 1)
    def _fin():
        idx_out_ref[...] = idx_sc[...]


def _gat_body(idx_ref, x_ref, w_ref, a_ref, b_ref, out_ref, rows_sc, sem):
    for i in range(M):
        pltpu.make_async_copy(
            x_ref.at[pl.ds(idx_ref[i], 1), :],
            rows_sc.at[pl.ds(i, 1), :], sem).start()
    for i in range(M):
        pltpu.make_async_copy(
            x_ref.at[pl.ds(0, 1), :],
            rows_sc.at[pl.ds(i, 1), :], sem).wait()

    sel = rows_sc[...]
    h = jnp.dot(sel, w_ref[...], preferred_element_type=jnp.float32)
    a0 = a_ref[0:1, :]
    a1 = a_ref[1:2, :]
    c = jnp.sum(h[0:1, :] * a0)
    d = jnp.sum(h * a1, axis=1, keepdims=True)
    lg = c + d
    lg = jnp.where(lg >= 0, lg, SLOPE * lg)
    ridx = jax.lax.broadcasted_iota(jnp.int32, (M, 1), 0)
    e = jnp.where(ridx >= 1, jnp.exp(lg), 0.0)
    alpha = e / jnp.sum(e)
    hp = jnp.sum(alpha * h, axis=0, keepdims=True) + b_ref[...]
    out_ref[...] = jnp.broadcast_to(hp, (NUM_HEAD, F))


def kernel(x, adj, weight, a, bias):
    idx2 = pl.pallas_call(
        _scan_body,
        grid=(NBLK,),
        in_specs=[pl.BlockSpec((M, BLK), lambda j: (0, j))],
        out_specs=pl.BlockSpec((M, 1), lambda j: (0, 0)),
        out_shape=jax.ShapeDtypeStruct((M, 1), jnp.int32),
        scratch_shapes=[pltpu.VMEM((M, 1), jnp.float32),
                        pltpu.VMEM((M, 1), jnp.int32)],
    )(adj)
    idx = idx2.reshape(M)

    out = pl.pallas_call(
        _gat_body,
        grid_spec=pltpu.PrefetchScalarGridSpec(
            num_scalar_prefetch=1,
            grid=(1,),
            in_specs=[
                pl.BlockSpec(memory_space=pl.ANY),
                pl.BlockSpec((F, F), lambda i, idx_ref: (0, 0)),
                pl.BlockSpec((2, F), lambda i, idx_ref: (0, 0)),
                pl.BlockSpec((1, F), lambda i, idx_ref: (0, 0)),
            ],
            out_specs=pl.BlockSpec((NUM_HEAD, F), lambda i, idx_ref: (0, 0)),
            scratch_shapes=[pltpu.VMEM((M, F), jnp.float32),
                            pltpu.SemaphoreType.DMA],
        ),
        out_shape=jax.ShapeDtypeStruct((NUM_HEAD, F), jnp.float32),
    )(idx, x, weight, a.reshape(2, F), bias.reshape(1, F))
    return out.reshape(NUM_HEAD * F)


# speedup vs baseline: 5.3832x; 1.0046x over previous
import jax
import jax.numpy as jnp
from jax.experimental import pallas as pl
from jax.experimental.pallas import tpu as pltpu

M = 65
N = 100000
F = 128
NUM_HEAD = 8
SLOPE = 0.2
BLK = 25088
NBLK = (N + BLK - 1) // BLK
INT_MAX = jnp.iinfo(jnp.int32).max


def _scan_body(adj_ref, idx_out_ref, max_sc, idx_sc):
    j = pl.program_id(0)

    @pl.when(j == 0)
    def _init():
        max_sc[...] = jnp.full((M, 1), -1.0, jnp.float32)
        idx_sc[...] = jnp.zeros((M, 1), jnp.int32)

    def _update(vals):
        cols = j * BLK + jax.lax.broadcasted_iota(jnp.int32, (M, BLK), 1)
        bmax = jnp.max(vals, axis=1, keepdims=True)
        bidx = jnp.min(jnp.where(vals == bmax, cols, INT_MAX),
                       axis=1, keepdims=True)
        better = bmax > max_sc[...]
        max_sc[...] = jnp.where(better, bmax, max_sc[...])
        idx_sc[...] = jnp.where(better, bidx, idx_sc[...])

    @pl.when(j < NBLK - 1)
    def _full():
        _update(adj_ref[...])

    @pl.when(j == NBLK - 1)
    def _tail():
        cols = j * BLK + jax.lax.broadcasted_iota(jnp.int32, (M, BLK), 1)
        _update(jnp.where(cols < N, adj_ref[...], -1.0))

    @pl.when(j == NBLK - 1)
    def _fin():
        idx_out_ref[...] = idx_sc[...]


def _gat_body(idx_ref, x_ref, w_ref, a_ref, b_ref, out_ref, rows_sc, sem):
    for i in range(M):
        pltpu.make_async_copy(
            x_ref.at[pl.ds(idx_ref[i], 1), :],
            rows_sc.at[pl.ds(i, 1), :], sem).start()
    pltpu.make_async_copy(
        x_ref.at[pl.ds(0, M), :], rows_sc, sem).wait()

    sel = rows_sc[...]
    h = jnp.dot(sel, w_ref[...], preferred_element_type=jnp.float32)
    a0 = a_ref[0:1, :]
    a1 = a_ref[1:2, :]
    c = jnp.sum(h[0:1, :] * a0)
    d = jnp.sum(h * a1, axis=1, keepdims=True)
    lg = c + d
    lg = jnp.where(lg >= 0, lg, SLOPE * lg)
    ridx = jax.lax.broadcasted_iota(jnp.int32, (M, 1), 0)
    e = jnp.where(ridx >= 1, jnp.exp(lg), 0.0)
    alpha = e / jnp.sum(e)
    hp = jnp.sum(alpha * h, axis=0, keepdims=True) + b_ref[...]
    out_ref[...] = jnp.broadcast_to(hp, (NUM_HEAD, F))


def kernel(x, adj, weight, a, bias):
    idx2 = pl.pallas_call(
        _scan_body,
        grid=(NBLK,),
        in_specs=[pl.BlockSpec((M, BLK), lambda j: (0, j))],
        out_specs=pl.BlockSpec((M, 1), lambda j: (0, 0)),
        out_shape=jax.ShapeDtypeStruct((M, 1), jnp.int32),
        scratch_shapes=[pltpu.VMEM((M, 1), jnp.float32),
                        pltpu.VMEM((M, 1), jnp.int32)],
    )(adj)
    idx = idx2.reshape(M)

    out = pl.pallas_call(
        _gat_body,
        grid_spec=pltpu.PrefetchScalarGridSpec(
            num_scalar_prefetch=1,
            grid=(1,),
            in_specs=[
                pl.BlockSpec(memory_space=pl.ANY),
                pl.BlockSpec((F, F), lambda i, idx_ref: (0, 0)),
                pl.BlockSpec((2, F), lambda i, idx_ref: (0, 0)),
                pl.BlockSpec((1, F), lambda i, idx_ref: (0, 0)),
            ],
            out_specs=pl.BlockSpec((NUM_HEAD, F), lambda i, idx_ref: (0, 0)),
            scratch_shapes=[pltpu.VMEM((M, F), jnp.float32),
                            pltpu.SemaphoreType.DMA],
        ),
        out_shape=jax.ShapeDtypeStruct((NUM_HEAD, F), jnp.float32),
    )(idx, x, weight, a.reshape(2, F), bias.reshape(1, F))
    return out.reshape(NUM_HEAD * F)
